# Initial kernel scaffold; baseline (speedup 1.0000x reference)
import jax
import jax.numpy as jnp
from jax.experimental import pallas as pl


def kernel(x, edge_index, W1, b1, W2, b2):
    raise NotImplementedError("write your pallas kernel here")


# speedup vs baseline: 35.4381x; 35.4381x over previous
import functools

import jax
import jax.numpy as jnp
from jax import lax
from jax.experimental import pallas as pl
from jax.experimental.pallas import tpu as pltpu
from jax.experimental.pallas import tpu_sc as plsc

N_NODES = 10000
N_EDGES = 320000
IN_D = 128
HID = 64

NC = 2
NS = 16
NW = NC * NS

NP = 10240
RPT = NP // NS
EPT = N_EDGES // NW
K = 80
NCHUNK = EPT // K
NBUF = 5
NGRP = NCHUNK // NBUF

_mesh = plsc.VectorSubcoreMesh(core_axis_name="c", subcore_axis_name="s")
_sc_params = pltpu.CompilerParams(use_tc_tiling_on_sc=False)


@functools.partial(
    pl.kernel,
    out_type=jax.ShapeDtypeStruct((NC * NP,), jnp.float32),
    mesh=_mesh,
    compiler_params=_sc_params,
    scratch_types=[
        pltpu.VMEM((NCHUNK, K), jnp.int32),
        pltpu.VMEM((RPT,), jnp.float32),
        pltpu.VMEM((K,), jnp.float32),
        pltpu.VMEM_SHARED((NP,), jnp.float32),
    ],
)
def _deg_kernel(dst3d_hbm, degp_hbm, didx_v, z_v, ones_v, deg_sh):
    c = lax.axis_index("c")
    s = lax.axis_index("s")
    wid = c * NS + s

    def fill_z(i, _):
        z_v[pl.ds(i * 16, 16)] = jnp.zeros((16,), jnp.float32)
        return 0

    lax.fori_loop(0, RPT // 16, fill_z, 0)

    def fill_o(i, _):
        ones_v[pl.ds(i * 16, 16)] = jnp.ones((16,), jnp.float32)
        return 0

    lax.fori_loop(0, K // 16, fill_o, 0)

    pltpu.sync_copy(z_v, deg_sh.at[pl.ds(s * RPT, RPT)])
    pltpu.sync_copy(dst3d_hbm.at[wid], didx_v)
    plsc.subcore_barrier()

    def body(ci, _):
        pltpu.sync_copy(ones_v, deg_sh.at[didx_v.at[ci]], add=True)
        return 0

    lax.fori_loop(0, NCHUNK, body, 0)
    plsc.subcore_barrier()
    pltpu.sync_copy(deg_sh.at[pl.ds(s * RPT, RPT)],
                    degp_hbm.at[pl.ds(c * NP + s * RPT, RPT)])


@functools.partial(
    pl.kernel,
    out_type=jax.ShapeDtypeStruct((NC * NP, HID), jnp.float32),
    mesh=_mesh,
    compiler_params=_sc_params,
    scratch_types=[
        pltpu.VMEM((NCHUNK, K), jnp.int32),
        pltpu.VMEM((NCHUNK, K), jnp.int32),
    ] + [pltpu.VMEM((K, HID), jnp.float32) for _ in range(NBUF)] + [
        pltpu.VMEM_SHARED((NP, HID), jnp.float32),
        pltpu.SemaphoreType.DMA,
    ],
)
def _edge_kernel(ht_hbm, src3d_hbm, dst3d_hbm, aggp_hbm,
                 sidx_v, didx_v, r0, r1, r2, r3, r4, agg_sh, sem):
    rows = (r0, r1, r2, r3, r4)
    c = lax.axis_index("c")
    s = lax.axis_index("s")
    wid = c * NS + s

    def fill_z(j, _):
        for l in range(HID // 16):
            r0[j, pl.ds(l * 16, 16)] = jnp.zeros((16,), jnp.float32)
        return 0

    lax.fori_loop(0, K, fill_z, 0)

    def zcp(j, _):
        pltpu.sync_copy(r0, agg_sh.at[pl.ds(s * RPT + j * K, K)])
        return 0

    lax.fori_loop(0, RPT // K, zcp, 0)

    pltpu.sync_copy(src3d_hbm.at[wid], sidx_v)
    pltpu.sync_copy(dst3d_hbm.at[wid], didx_v)
    plsc.subcore_barrier()

    def grp(g, _):
        cps = []
        for b in range(NBUF):
            ci = g * NBUF + b
            cps.append(pltpu.async_copy(
                ht_hbm.at[sidx_v.at[ci]], rows[b], sem))
        for b in range(NBUF):
            cps[b].wait()
            pltpu.sync_copy(rows[b], agg_sh.at[didx_v.at[g * NBUF + b]],
                            add=True)
        return 0

    lax.fori_loop(0, NGRP, grp, 0)
    plsc.subcore_barrier()
    pltpu.sync_copy(agg_sh.at[pl.ds(s * RPT, RPT)],
                    aggp_hbm.at[pl.ds(c * NP + s * RPT, RPT)])


BR = 640


def _mm1_body(x_ref, w_ref, o_ref):
    o_ref[...] = jnp.dot(x_ref[...], w_ref[...],
                         preferred_element_type=jnp.float32)


def _scale_body(z_ref, dp_ref, o_ref):
    deg = dp_ref[0] + dp_ref[1] + 1.0
    dis = lax.rsqrt(deg)
    o_ref[...] = z_ref[...] * dis


def _mm2_body(ap_ref, z1_ref, dp_ref, b1_ref, w2_ref, z2_ref, ht2_ref):
    deg = dp_ref[0] + dp_ref[1] + 1.0
    dis = lax.rsqrt(deg)
    h = jnp.maximum(dis * (ap_ref[0] + ap_ref[1])
                    + (dis * dis) * z1_ref[...] + b1_ref[...], 0.0)
    z2 = jnp.dot(h, w2_ref[...], preferred_element_type=jnp.float32)
    z2_ref[...] = z2
    ht2_ref[...] = z2 * dis


def _fin_body(ap_ref, z2_ref, dp_ref, b2_ref, o_ref):
    deg = dp_ref[0] + dp_ref[1] + 1.0
    dis = lax.rsqrt(deg)
    o_ref[...] = (dis * (ap_ref[0] + ap_ref[1])
                  + (dis * dis) * z2_ref[...] + b2_ref[...])


def _row_spec(d):
    return pl.BlockSpec((BR, d), lambda i: (i, 0))


def _pair_spec(d):
    return pl.BlockSpec((NC, BR, d), lambda i: (0, i, 0))


def _full_spec(a, b):
    return pl.BlockSpec((a, b), lambda i: (0, 0))


def kernel(x, edge_index, W1, b1, W2, b2):
    src3d = edge_index[0].reshape(NW, NCHUNK, K)
    dst3d = edge_index[1].reshape(NW, NCHUNK, K)
    x_p = jnp.pad(x, ((0, NP - N_NODES), (0, 0)))

    degp = _deg_kernel(dst3d)
    degp3 = degp.reshape(NC, NP, 1)

    z1 = pl.pallas_call(
        _mm1_body,
        grid=(NP // BR,),
        in_specs=[_row_spec(IN_D), _full_spec(IN_D, HID)],
        out_specs=_row_spec(HID),
        out_shape=jax.ShapeDtypeStruct((NP, HID), jnp.float32),
    )(x_p, W1)

    ht1 = pl.pallas_call(
        _scale_body,
        grid=(NP // BR,),
        in_specs=[_row_spec(HID), _pair_spec(1)],
        out_specs=_row_spec(HID),
        out_shape=jax.ShapeDtypeStruct((NP, HID), jnp.float32),
    )(z1, degp3)

    aggp1 = _edge_kernel(ht1, src3d, dst3d).reshape(NC, NP, HID)

    z2, ht2 = pl.pallas_call(
        _mm2_body,
        grid=(NP // BR,),
        in_specs=[_pair_spec(HID), _row_spec(HID), _pair_spec(1),
                  _full_spec(1, HID), _full_spec(HID, HID)],
        out_specs=[_row_spec(HID), _row_spec(HID)],
        out_shape=[jax.ShapeDtypeStruct((NP, HID), jnp.float32),
                   jax.ShapeDtypeStruct((NP, HID), jnp.float32)],
    )(aggp1, z1, degp3, b1.reshape(1, HID), W2)

    aggp2 = _edge_kernel(ht2, src3d, dst3d).reshape(NC, NP, HID)

    out = pl.pallas_call(
        _fin_body,
        grid=(NP // BR,),
        in_specs=[_pair_spec(HID), _row_spec(HID), _pair_spec(1),
                  _full_spec(1, HID)],
        out_specs=_row_spec(HID),
        out_shape=jax.ShapeDtypeStruct((NP, HID), jnp.float32),
    )(aggp2, z2, degp3, b2.reshape(1, HID))

    return out[:N_NODES]


# speedup vs baseline: 36.3014x; 1.0244x over previous
import functools

import jax
import jax.numpy as jnp
from jax import lax
from jax.experimental import pallas as pl
from jax.experimental.pallas import tpu as pltpu
from jax.experimental.pallas import tpu_sc as plsc

N_NODES = 10000
N_EDGES = 320000
IN_D = 128
HID = 64

NC = 2
NS = 16
NW = NC * NS

NP = 10240
RPT = NP // NS
EPT = N_EDGES // NW
K = 80
NCHUNK = EPT // K
NBUF = 5
NGRP = NCHUNK // NBUF

_mesh = plsc.VectorSubcoreMesh(core_axis_name="c", subcore_axis_name="s")
_sc_params = pltpu.CompilerParams(use_tc_tiling_on_sc=False)


@functools.partial(
    pl.kernel,
    out_type=jax.ShapeDtypeStruct((NC * NP,), jnp.float32),
    mesh=_mesh,
    compiler_params=_sc_params,
    scratch_types=[
        pltpu.VMEM((NCHUNK, K), jnp.int32),
        pltpu.VMEM((RPT,), jnp.float32),
        pltpu.VMEM((K,), jnp.float32),
        pltpu.VMEM_SHARED((NP,), jnp.float32),
    ],
)
def _deg_kernel(dst3d_hbm, degp_hbm, didx_v, z_v, ones_v, deg_sh):
    c = lax.axis_index("c")
    s = lax.axis_index("s")
    wid = c * NS + s

    def fill_z(i, _):
        z_v[pl.ds(i * 16, 16)] = jnp.zeros((16,), jnp.float32)
        return 0

    lax.fori_loop(0, RPT // 16, fill_z, 0)

    def fill_o(i, _):
        ones_v[pl.ds(i * 16, 16)] = jnp.ones((16,), jnp.float32)
        return 0

    lax.fori_loop(0, K // 16, fill_o, 0)

    pltpu.sync_copy(z_v, deg_sh.at[pl.ds(s * RPT, RPT)])
    pltpu.sync_copy(dst3d_hbm.at[wid], didx_v)
    plsc.subcore_barrier()

    def body(ci, _):
        pltpu.sync_copy(ones_v, deg_sh.at[didx_v.at[ci]], add=True)
        return 0

    lax.fori_loop(0, NCHUNK, body, 0)
    plsc.subcore_barrier()
    pltpu.sync_copy(deg_sh.at[pl.ds(s * RPT, RPT)],
                    degp_hbm.at[pl.ds(c * NP + s * RPT, RPT)])


@functools.partial(
    pl.kernel,
    out_type=jax.ShapeDtypeStruct((NC * NP, HID), jnp.float32),
    mesh=_mesh,
    compiler_params=_sc_params,
    scratch_types=[
        pltpu.VMEM((NCHUNK, K), jnp.int32),
        pltpu.VMEM((NCHUNK, K), jnp.int32),
    ] + [pltpu.VMEM((K, HID), jnp.float32) for _ in range(NBUF)] + [
        pltpu.VMEM_SHARED((NP, HID), jnp.float32),
        pltpu.SemaphoreType.DMA,
        pltpu.SemaphoreType.DMA,
    ],
)
def _edge_kernel(ht_hbm, src3d_hbm, dst3d_hbm, aggp_hbm,
                 sidx_v, didx_v, r0, r1, r2, r3, r4, agg_sh, sem_g, sem_s):
    rows = (r0, r1, r2, r3, r4)
    c = lax.axis_index("c")
    s = lax.axis_index("s")
    wid = c * NS + s

    def fill_z(j, _):
        for l in range(HID // 16):
            r0[j, pl.ds(l * 16, 16)] = jnp.zeros((16,), jnp.float32)
        return 0

    lax.fori_loop(0, K, fill_z, 0)

    def zcp(j, _):
        pltpu.sync_copy(r0, agg_sh.at[pl.ds(s * RPT + j * K, K)])
        return 0

    lax.fori_loop(0, RPT // K, zcp, 0)

    pltpu.sync_copy(src3d_hbm.at[wid], sidx_v)
    pltpu.sync_copy(dst3d_hbm.at[wid], didx_v)
    plsc.subcore_barrier()

    def grp(g, _):
        @pl.when(g > 0)
        def _drain():
            for b in range(NBUF):
                pltpu.make_async_copy(ht_hbm.at[sidx_v.at[0]], rows[b],
                                      sem_s).wait()

        cps = []
        for b in range(NBUF):
            ci = g * NBUF + b
            cps.append(pltpu.async_copy(
                ht_hbm.at[sidx_v.at[ci]], rows[b], sem_g))
        for b in range(NBUF):
            cps[b].wait()
            pltpu.async_copy(rows[b], agg_sh.at[didx_v.at[g * NBUF + b]],
                             sem_s, add=True)
        return 0

    lax.fori_loop(0, NGRP, grp, 0)
    for b in range(NBUF):
        pltpu.make_async_copy(ht_hbm.at[sidx_v.at[0]], rows[b], sem_s).wait()
    plsc.subcore_barrier()
    pltpu.sync_copy(agg_sh.at[pl.ds(s * RPT, RPT)],
                    aggp_hbm.at[pl.ds(c * NP + s * RPT, RPT)])


BR = 640


def _mm1_body(x_ref, w_ref, o_ref):
    o_ref[...] = jnp.dot(x_ref[...], w_ref[...],
                         preferred_element_type=jnp.float32)


def _scale_body(z_ref, dp_ref, o_ref):
    deg = dp_ref[0] + dp_ref[1] + 1.0
    dis = lax.rsqrt(deg)
    o_ref[...] = z_ref[...] * dis


def _mm2_body(ap_ref, z1_ref, dp_ref, b1_ref, w2_ref, z2_ref, ht2_ref):
    deg = dp_ref[0] + dp_ref[1] + 1.0
    dis = lax.rsqrt(deg)
    h = jnp.maximum(dis * (ap_ref[0] + ap_ref[1])
                    + (dis * dis) * z1_ref[...] + b1_ref[...], 0.0)
    z2 = jnp.dot(h, w2_ref[...], preferred_element_type=jnp.float32)
    z2_ref[...] = z2
    ht2_ref[...] = z2 * dis


def _fin_body(ap_ref, z2_ref, dp_ref, b2_ref, o_ref):
    deg = dp_ref[0] + dp_ref[1] + 1.0
    dis = lax.rsqrt(deg)
    o_ref[...] = (dis * (ap_ref[0] + ap_ref[1])
                  + (dis * dis) * z2_ref[...] + b2_ref[...])


def _row_spec(d):
    return pl.BlockSpec((BR, d), lambda i: (i, 0))


def _pair_spec(d):
    return pl.BlockSpec((NC, BR, d), lambda i: (0, i, 0))


def _full_spec(a, b):
    return pl.BlockSpec((a, b), lambda i: (0, 0))


def kernel(x, edge_index, W1, b1, W2, b2):
    src3d = edge_index[0].reshape(NW, NCHUNK, K)
    dst3d = edge_index[1].reshape(NW, NCHUNK, K)
    x_p = jnp.pad(x, ((0, NP - N_NODES), (0, 0)))

    degp = _deg_kernel(dst3d)
    degp3 = degp.reshape(NC, NP, 1)

    z1 = pl.pallas_call(
        _mm1_body,
        grid=(NP // BR,),
        in_specs=[_row_spec(IN_D), _full_spec(IN_D, HID)],
        out_specs=_row_spec(HID),
        out_shape=jax.ShapeDtypeStruct((NP, HID), jnp.float32),
    )(x_p, W1)

    ht1 = pl.pallas_call(
        _scale_body,
        grid=(NP // BR,),
        in_specs=[_row_spec(HID), _pair_spec(1)],
        out_specs=_row_spec(HID),
        out_shape=jax.ShapeDtypeStruct((NP, HID), jnp.float32),
    )(z1, degp3)

    aggp1 = _edge_kernel(ht1, src3d, dst3d).reshape(NC, NP, HID)

    z2, ht2 = pl.pallas_call(
        _mm2_body,
        grid=(NP // BR,),
        in_specs=[_pair_spec(HID), _row_spec(HID), _pair_spec(1),
                  _full_spec(1, HID), _full_spec(HID, HID)],
        out_specs=[_row_spec(HID), _row_spec(HID)],
        out_shape=[jax.ShapeDtypeStruct((NP, HID), jnp.float32),
                   jax.ShapeDtypeStruct((NP, HID), jnp.float32)],
    )(aggp1, z1, degp3, b1.reshape(1, HID), W2)

    aggp2 = _edge_kernel(ht2, src3d, dst3d).reshape(NC, NP, HID)

    out = pl.pallas_call(
        _fin_body,
        grid=(NP // BR,),
        in_specs=[_pair_spec(HID), _row_spec(HID), _pair_spec(1),
                  _full_spec(1, HID)],
        out_specs=_row_spec(HID),
        out_shape=jax.ShapeDtypeStruct((NP, HID), jnp.float32),
    )(aggp2, z2, degp3, b2.reshape(1, HID))

    return out[:N_NODES]
